# baseline (speedup 1.0000x reference)
import jax
import jax.numpy as jnp
from jax import lax
from jax.experimental import pallas as pl
from jax.experimental.pallas import tpu as pltpu
from jax.experimental.pallas import tpu_sc as plsc

NF = 26
V = 100000
B = 16384
NC, NS, L = 2, 16, 16
HB = B // 2
RPT = HB // NS
TCH = 128
SUB = 32
GCH = 1024
RCH = 128

_CP = pltpu.CompilerParams(needs_layout_passes=False)
_MESH = dict(core_axis_name="c", subcore_axis_name="s",
             num_cores=NC, num_subcores=NS)


def _main_body(x_hbm, t2d_hbm, out2_hbm,
               xT_sh, trow, bufA, bufB, idxc, red, outq, sem):
    c_id = lax.axis_index("c")
    s_id = lax.axis_index("s")
    wid = s_id * NC + c_id
    iota = lax.iota(jnp.int32, L)

    @pl.when(wid < NF)
    def _():
        pltpu.async_copy(t2d_hbm.at[wid], trow, sem)

    for h in range(2):
        def chunk_a(k, _):
            r0 = h * HB + s_id * RPT + k * TCH
            for sub in range(TCH // SUB):
                pltpu.sync_copy(x_hbm.at[pl.ds(r0 + sub * SUB, SUB)], bufA)

                @plsc.parallel_loop(0, NF * (SUB // L), step=1, unroll=8)
                def _(v):
                    f = v // (SUB // L)
                    j = v % (SUB // L)
                    g = plsc.load_gather(bufA, [iota + j * L, jnp.zeros((L,), jnp.int32) + f])
                    bufB[f, pl.ds(sub * SUB + j * L, L)] = plsc.bitcast(g, jnp.float32)
            pltpu.sync_copy(bufB, xT_sh.at[:, pl.ds(s_id * RPT + k * TCH, TCH)])
            return 0
        lax.fori_loop(0, RPT // TCH, chunk_a, 0)
        plsc.subcore_barrier()

        @pl.when(wid < NF)
        def _():
            if h == 0:
                pltpu.make_async_copy(t2d_hbm.at[wid], trow, sem).wait()

            def chunk_b(cb, _):
                lbase = cb * GCH
                pltpu.sync_copy(xT_sh.at[wid, pl.ds(lbase, GCH)], idxc)

                @plsc.parallel_loop(0, GCH, step=L, unroll=8)
                def _(off):
                    ids = plsc.bitcast(idxc[pl.ds(off, L)], jnp.int32)
                    idxc[pl.ds(off, L)] = plsc.load_gather(trow, [ids])
                pltpu.sync_copy(idxc, xT_sh.at[wid, pl.ds(lbase, GCH)])
                return 0
            lax.fori_loop(0, HB // GCH, chunk_b, 0)
        plsc.subcore_barrier()

        def chunk_c(q, _):
            col0 = s_id * RPT + q * RCH
            pltpu.sync_copy(xT_sh.at[:, pl.ds(col0, RCH)], red)

            def rv(v, _):
                acc = red[c_id, pl.ds(v * L, L)]
                for j in range(1, 13):
                    acc = acc + red[2 * j + c_id, pl.ds(v * L, L)]
                outq[pl.ds(q * RCH + v * L, L)] = acc
                return 0
            lax.fori_loop(0, RCH // L, rv, 0)
            return 0
        lax.fori_loop(0, RPT // RCH, chunk_c, 0)
        pltpu.sync_copy(outq, out2_hbm.at[c_id, pl.ds(h * HB + s_id * RPT, RPT)])
        if h == 0:
            plsc.subcore_barrier()


def _tc_add_body(p_ref, o_ref):
    o_ref[...] = p_ref[0, :] + p_ref[1, :]


def kernel(X, table):
    t2d = table.reshape(NF, V)

    out2 = pl.kernel(
        _main_body,
        out_type=jax.ShapeDtypeStruct((NC, B), jnp.float32),
        mesh=plsc.VectorSubcoreMesh(**_MESH),
        scratch_types=[
            pltpu.VMEM_SHARED((32, HB), jnp.float32),
            pltpu.VMEM((V,), jnp.float32),
            pltpu.VMEM((SUB, NF), jnp.int32),
            pltpu.VMEM((32, TCH), jnp.float32),
            pltpu.VMEM((GCH,), jnp.float32),
            pltpu.VMEM((32, RCH), jnp.float32),
            pltpu.VMEM((RPT,), jnp.float32),
            pltpu.SemaphoreType.DMA,
        ],
        compiler_params=_CP,
    )(X, t2d)

    out = pl.pallas_call(
        _tc_add_body,
        out_shape=jax.ShapeDtypeStruct((B,), jnp.float32),
    )(out2)
    return out.reshape(B, 1)


# speedup vs baseline: 1.7473x; 1.7473x over previous
import jax
import jax.numpy as jnp
from jax import lax
from jax.experimental import pallas as pl
from jax.experimental.pallas import tpu as pltpu
from jax.experimental.pallas import tpu_sc as plsc

NF = 26
V = 100000
B = 16384
NC, NS, L = 2, 16, 16
HB = B // 2
RPT = HB // NS
GCH = 2048
NCH = HB // GCH
RCH = 128

_CP = pltpu.CompilerParams(needs_layout_passes=False)
_MESH = dict(core_axis_name="c", subcore_axis_name="s",
             num_cores=NC, num_subcores=NS)


def _main_body(xt_hbm, t2d_hbm, out2_hbm,
               val_sh, trow, idx0, idx1, val0, val1, red, outq, tsem, isem, osem):
    c_id = lax.axis_index("c")
    s_id = lax.axis_index("s")
    wid = s_id * NC + c_id

    @pl.when(wid < NF)
    def _():
        pltpu.async_copy(t2d_hbm.at[wid], trow, tsem)

    idxs = (idx0, idx1)
    vals = (val0, val1)

    for h in range(2):
        @pl.when(wid < NF)
        def _():
            def in_cp(cb):
                return pltpu.make_async_copy(
                    xt_hbm.at[wid, pl.ds(h * HB + cb * GCH, GCH)], idxs[cb % 2], isem)

            def out_cp(cb):
                return pltpu.make_async_copy(
                    vals[cb % 2], val_sh.at[wid, pl.ds(cb * GCH, GCH)], osem)

            pltpu.async_copy(
                xt_hbm.at[wid, pl.ds(h * HB, GCH)], idxs[0], isem)
            if h == 0:
                pltpu.make_async_copy(t2d_hbm.at[wid], trow, tsem).wait()
            for cb in range(NCH):
                in_cp(cb).wait()
                if cb >= 2:
                    out_cp(cb - 2).wait()
                if cb + 1 < NCH:
                    pltpu.async_copy(
                        xt_hbm.at[wid, pl.ds(h * HB + (cb + 1) * GCH, GCH)],
                        idxs[(cb + 1) % 2], isem)
                ib, vb = idxs[cb % 2], vals[cb % 2]

                @plsc.parallel_loop(0, GCH, step=L, unroll=8)
                def _(off):
                    vb[pl.ds(off, L)] = plsc.load_gather(trow, [ib[pl.ds(off, L)]])
                pltpu.async_copy(vb, val_sh.at[wid, pl.ds(cb * GCH, GCH)], osem)
            out_cp(NCH - 2).wait()
            out_cp(NCH - 1).wait()
        plsc.subcore_barrier()

        def chunk_c(q, _):
            col0 = s_id * RPT + q * RCH
            pltpu.sync_copy(val_sh.at[:, pl.ds(col0, RCH)], red)

            def rv(v, _):
                acc = red[c_id, pl.ds(v * L, L)]
                for j in range(1, 13):
                    acc = acc + red[2 * j + c_id, pl.ds(v * L, L)]
                outq[pl.ds(q * RCH + v * L, L)] = acc
                return 0
            lax.fori_loop(0, RCH // L, rv, 0)
            return 0
        lax.fori_loop(0, RPT // RCH, chunk_c, 0)
        pltpu.sync_copy(outq, out2_hbm.at[c_id, pl.ds(h * HB + s_id * RPT, RPT)])
        if h == 0:
            plsc.subcore_barrier()


def _tc_add_body(p_ref, o_ref):
    o_ref[...] = p_ref[0, :] + p_ref[1, :]


def kernel(X, table):
    xt = X.T
    t2d = table.reshape(NF, V)

    out2 = pl.kernel(
        _main_body,
        out_type=jax.ShapeDtypeStruct((NC, B), jnp.float32),
        mesh=plsc.VectorSubcoreMesh(**_MESH),
        scratch_types=[
            pltpu.VMEM_SHARED((32, HB), jnp.float32),
            pltpu.VMEM((V,), jnp.float32),
            pltpu.VMEM((GCH,), jnp.int32),
            pltpu.VMEM((GCH,), jnp.int32),
            pltpu.VMEM((GCH,), jnp.float32),
            pltpu.VMEM((GCH,), jnp.float32),
            pltpu.VMEM((32, RCH), jnp.float32),
            pltpu.VMEM((RPT,), jnp.float32),
            pltpu.SemaphoreType.DMA,
            pltpu.SemaphoreType.DMA,
            pltpu.SemaphoreType.DMA,
        ],
        compiler_params=_CP,
    )(xt, t2d)

    out = pl.pallas_call(
        _tc_add_body,
        out_shape=jax.ShapeDtypeStruct((B,), jnp.float32),
    )(out2)
    return out.reshape(B, 1)
